# baseline (speedup 1.0000x reference)
import functools

import jax
import jax.numpy as jnp
from jax import lax
from jax.experimental import pallas as pl
from jax.experimental.pallas import tpu as pltpu
from jax.experimental.pallas import tpu_sc as plsc

N = 500000
D = 128
Z = 87
ZP = 88
E = 20
EP = 32

NC = 2
NS = 16
NW = NC * NS

C = 80
N_CHUNKS = N // C
BASE_CHUNKS = N_CHUNKS // NW
EXTRA = N_CHUNKS % NW
TOTAL = BASE_CHUNKS + 1
NBUF = 4

_mesh = plsc.VectorSubcoreMesh(core_axis_name="c", subcore_axis_name="s")


@functools.partial(
    pl.kernel,
    out_type=jax.ShapeDtypeStruct((N, D), jnp.float32),
    mesh=_mesh,
    scratch_types=[
        pltpu.VMEM((NBUF, C), jnp.int32),
        pltpu.VMEM((NBUF, C, D), jnp.float32),
        pltpu.VMEM((8, EP), jnp.float32),
        pltpu.VMEM((E, D), jnp.float32),
        pltpu.VMEM((8, D), jnp.float32),
        pltpu.VMEM_SHARED((ZP, D), jnp.float32),
        [pltpu.SemaphoreType.DMA] * NBUF,
        [pltpu.SemaphoreType.DMA] * NBUF,
        [pltpu.SemaphoreType.DMA] * NBUF,
    ],
)
def _embed_kernel(idx_hbm, ec_hbm, clt_hbm, emb_hbm, out_hbm,
                  idx_v, rows_v, ec_v, clt_v, fused_v, table_sh,
                  si, sg, sw):
    sid = lax.axis_index("s")
    wid = sid * NC + lax.axis_index("c")
    n_chunks = BASE_CHUNKS + jnp.where(wid < EXTRA, 1, 0)
    last = wid + (n_chunks - 1) * NW

    z0 = pl.multiple_of(jnp.minimum(sid * 8, ZP - 8), 8)
    pltpu.sync_copy(ec_hbm.at[pl.ds(z0, 8)], ec_v)
    pltpu.sync_copy(clt_hbm, clt_v)
    pltpu.sync_copy(emb_hbm.at[pl.ds(z0, 8)], fused_v)

    lane = lax.iota(jnp.int32, 16)

    def row_body(r, carry):
        ea = ec_v[r, pl.ds(0, 16)]
        eb = ec_v[r, pl.ds(16, 16)]
        for fb in range(D // 16):
            acc = fused_v[r, pl.ds(16 * fb, 16)]
            for e in range(E):
                src = ea if e < 16 else eb
                se = jnp.take(src, lane * 0 + (e % 16))
                acc = acc + se * clt_v[e, pl.ds(16 * fb, 16)]
            fused_v[r, pl.ds(16 * fb, 16)] = acc
        return carry

    lax.fori_loop(0, 8, row_body, 0)
    pltpu.sync_copy(fused_v, table_sh.at[pl.ds(z0, 8)])
    plsc.subcore_barrier()

    def off(i):
        return jnp.minimum(wid + i * NW, last) * C

    def _wait_idx(s):
        pltpu.make_async_copy(idx_hbm.at[pl.ds(0, C)], idx_v.at[s], si[s]).wait()

    def _wait_write(s):
        pltpu.make_async_copy(rows_v.at[s], out_hbm.at[pl.ds(0, C)], sw[s]).wait()

    for s in range(NBUF):
        pltpu.async_copy(idx_hbm.at[pl.ds(off(s), C)], idx_v.at[s], si[s])

    for s in range(2):
        _wait_idx(s)
        pltpu.async_copy(table_sh.at[idx_v.at[s]], rows_v.at[s], sg[s])

    def quad(q, carry):
        for s in range(NBUF):
            i = NBUF * q + s
            nxt = (s + 2) % NBUF

            @pl.when(i + 2 < TOTAL)
            def _():
                @pl.when((q > 0) | (s >= NBUF - 2))
                def _():
                    _wait_write(nxt)

                _wait_idx(nxt)
                pltpu.async_copy(
                    table_sh.at[idx_v.at[nxt]], rows_v.at[nxt], sg[nxt]
                )

            pltpu.make_async_copy(
                table_sh.at[idx_v.at[s]], rows_v.at[s], sg[s]
            ).wait()
            pltpu.async_copy(rows_v.at[s], out_hbm.at[pl.ds(off(i), C)], sw[s])

            @pl.when(i + NBUF < TOTAL)
            def _():
                pltpu.async_copy(
                    idx_hbm.at[pl.ds(off(i + NBUF), C)], idx_v.at[s], si[s]
                )
        return carry

    lax.fori_loop(0, TOTAL // NBUF, quad, 0)

    for s in range(NBUF):
        pltpu.make_async_copy(rows_v.at[s], out_hbm.at[pl.ds(0, C)], sw[s]).wait()


def kernel(atomic_numbers, electron_config, emb_table, config_linear):
    ec_pad = jnp.zeros((ZP, EP), jnp.float32).at[:Z, :E].set(electron_config)
    emb_pad = jnp.zeros((ZP, D), jnp.float32).at[:Z].set(emb_table)
    return _embed_kernel(
        atomic_numbers.astype(jnp.int32), ec_pad, config_linear.T, emb_pad
    )


# speedup vs baseline: 1.0401x; 1.0401x over previous
import functools

import jax
import jax.numpy as jnp
from jax import lax
from jax.experimental import pallas as pl
from jax.experimental.pallas import tpu as pltpu
from jax.experimental.pallas import tpu_sc as plsc

N = 500000
D = 128
Z = 87

NC = 2
NS = 16
NW = NC * NS

C = 80
N_CHUNKS = N // C
BASE_CHUNKS = N_CHUNKS // NW
EXTRA = N_CHUNKS % NW
TOTAL = BASE_CHUNKS + 1
NIDX = 4


def _table_body(ec_ref, cl_ref, emb_ref, out_ref):
    out_ref[...] = emb_ref[...] + lax.dot_general(
        ec_ref[...],
        cl_ref[...],
        (((1,), (1,)), ((), ())),
        preferred_element_type=jnp.float32,
    )


def _build_table(electron_config, cl, emb_table):
    return pl.pallas_call(
        _table_body,
        out_shape=jax.ShapeDtypeStruct((Z, D), jnp.float32),
    )(electron_config, cl, emb_table)


_mesh = plsc.VectorSubcoreMesh(core_axis_name="c", subcore_axis_name="s")


@functools.partial(
    pl.kernel,
    out_type=jax.ShapeDtypeStruct((N, D), jnp.float32),
    mesh=_mesh,
    scratch_types=[
        pltpu.VMEM((NIDX, C), jnp.int32),
        pltpu.VMEM((NIDX, C, D), jnp.float32),
        pltpu.VMEM_SHARED((Z, D), jnp.float32),
        [pltpu.SemaphoreType.DMA] * NIDX,
        [pltpu.SemaphoreType.DMA] * NIDX,
        [pltpu.SemaphoreType.DMA] * NIDX,
    ],
)
def _gather_kernel(idx_hbm, table_hbm, out_hbm, idx_v, rows_v, table_sh,
                   si, sg, sw):
    sid = lax.axis_index("s")
    wid = sid * NC + lax.axis_index("c")
    n_chunks = BASE_CHUNKS + jnp.where(wid < EXTRA, 1, 0)
    last = wid + (n_chunks - 1) * NW

    def off(i):
        return jnp.minimum(wid + i * NW, last) * C

    for s in range(NIDX):
        pltpu.async_copy(idx_hbm.at[pl.ds(off(s), C)], idx_v.at[s], si[s])

    @pl.when(sid == 0)
    def _stage():
        pltpu.sync_copy(table_hbm, table_sh)

    plsc.subcore_barrier()

    def _wait_idx(s):
        pltpu.make_async_copy(idx_hbm.at[pl.ds(0, C)], idx_v.at[s], si[s]).wait()

    def _wait_write(s):
        pltpu.make_async_copy(rows_v.at[s], out_hbm.at[pl.ds(0, C)], sw[s]).wait()

    for s in range(2):
        _wait_idx(s)
        pltpu.async_copy(table_sh.at[idx_v.at[s]], rows_v.at[s], sg[s])

    def quad(q, carry):
        for s in range(NIDX):
            i = NIDX * q + s
            nxt = (s + 2) % NIDX

            @pl.when(i + 2 < TOTAL)
            def _():
                @pl.when((q > 0) | (s >= NIDX - 2))
                def _():
                    _wait_write(nxt)

                _wait_idx(nxt)
                pltpu.async_copy(
                    table_sh.at[idx_v.at[nxt]], rows_v.at[nxt], sg[nxt]
                )

            pltpu.make_async_copy(
                table_sh.at[idx_v.at[s]], rows_v.at[s], sg[s]
            ).wait()
            pltpu.async_copy(rows_v.at[s], out_hbm.at[pl.ds(off(i), C)], sw[s])

            @pl.when(i + NIDX < TOTAL)
            def _():
                pltpu.async_copy(
                    idx_hbm.at[pl.ds(off(i + NIDX), C)], idx_v.at[s], si[s]
                )
        return carry

    lax.fori_loop(0, TOTAL // NIDX, quad, 0)

    for s in range(NIDX):
        pltpu.make_async_copy(rows_v.at[s], out_hbm.at[pl.ds(0, C)], sw[s]).wait()


def kernel(atomic_numbers, electron_config, emb_table, config_linear):
    table = _build_table(electron_config, config_linear, emb_table)
    return _gather_kernel(atomic_numbers.astype(jnp.int32), table)
